# baseline (speedup 1.0000x reference)
import functools
import jax
import jax.numpy as jnp
from jax import lax
from jax.experimental import pallas as pl
from jax.experimental.pallas import tpu as pltpu
from jax.experimental.pallas import tpu_sc as plsc

VOCAB = 151936
DIM = 1152
B = 8
S = 2048
NTOK = B * S


@functools.lru_cache(maxsize=None)
def _build_gather():
    info = plsc.get_sparse_core_info()
    nc, ns = info.num_cores, info.num_subcores
    nw = nc * ns
    rows_per_w = NTOK // nw
    chunk = 8
    nbuf = 8
    nchunk = rows_per_w // chunk
    nround = nchunk // nbuf

    w_per_b = S // rows_per_w

    mesh = plsc.VectorSubcoreMesh(core_axis_name="c", subcore_axis_name="s")

    @functools.partial(
        pl.kernel,
        mesh=mesh,
        out_type=jax.ShapeDtypeStruct((B, S, DIM), jnp.float32),
        scratch_types=[
            pltpu.VMEM((rows_per_w,), jnp.int32),
        ]
        + [pltpu.VMEM((chunk, DIM), jnp.float32) for _ in range(nbuf)]
        + [pltpu.SemaphoreType.DMA for _ in range(2 * nbuf)],
    )
    def gather(idx_hbm, table_hbm, out_hbm, idx_v, *bufs_and_sems):
        bufs = bufs_and_sems[:nbuf]
        sem_g = bufs_and_sems[nbuf:2 * nbuf]
        sem_o = bufs_and_sems[2 * nbuf:]
        wid = lax.axis_index("s") * nc + lax.axis_index("c")
        brow = wid // w_per_b
        scol = (wid % w_per_b) * rows_per_w
        head = 2 * nbuf * chunk
        pltpu.sync_copy(idx_hbm.at[brow, pl.ds(scol, head)],
                        idx_v.at[pl.ds(0, head)])

        def gather_copy(c, p):
            return pltpu.make_async_copy(
                table_hbm.at[idx_v.at[pl.ds(c * chunk, chunk)]],
                bufs[p], sem_g[p])

        def out_copy(c, p):
            return pltpu.make_async_copy(
                bufs[p], out_hbm.at[brow, pl.ds(scol + c * chunk, chunk)],
                sem_o[p])

        for b in range(nbuf):
            gather_copy(b, b).start()
        pltpu.sync_copy(
            idx_hbm.at[brow, pl.ds(scol + head, rows_per_w - head)],
            idx_v.at[pl.ds(head, rows_per_w - head)])

        def round_body(i, carry):
            g0 = i * nbuf
            for b in range(nbuf):
                c = g0 + b
                gather_copy(c, b).wait()
                out_copy(c, b).start()
                out_copy(c, b).wait()
                gather_copy(c + nbuf, b).start()
            return carry

        lax.fori_loop(0, nround - 1, round_body, 0)

        g0 = (nround - 1) * nbuf
        for b in range(nbuf):
            gather_copy(g0 + b, b).wait()
            out_copy(g0 + b, b).start()
        for b in range(nbuf):
            out_copy(g0 + b, b).wait()

    return gather


def kernel(x, emb_weight):
    return _build_gather()(x.astype(jnp.int32), emb_weight)


# speedup vs baseline: 1.0119x; 1.0119x over previous
import functools
import jax
import jax.numpy as jnp
from jax import lax
from jax.experimental import pallas as pl
from jax.experimental.pallas import tpu as pltpu
from jax.experimental.pallas import tpu_sc as plsc

VOCAB = 151936
DIM = 1152
B = 8
S = 2048
NTOK = B * S


@functools.lru_cache(maxsize=None)
def _build_gather():
    info = plsc.get_sparse_core_info()
    nc, ns = info.num_cores, info.num_subcores
    nw = nc * ns
    rows_per_w = NTOK // nw
    chunk = 8
    nbuf = 8
    nchunk = rows_per_w // chunk
    nround = nchunk // nbuf

    w_per_b = S // rows_per_w

    mesh = plsc.VectorSubcoreMesh(core_axis_name="c", subcore_axis_name="s")

    @functools.partial(
        pl.kernel,
        mesh=mesh,
        out_type=jax.ShapeDtypeStruct((B, S, DIM), jnp.float32),
        scratch_types=[
            pltpu.VMEM((rows_per_w,), jnp.int32),
        ]
        + [pltpu.VMEM((chunk, DIM), jnp.float32) for _ in range(nbuf)]
        + [pltpu.SemaphoreType.DMA for _ in range(2 * nbuf)],
    )
    def gather(idx_hbm, table_hbm, out_hbm, idx_v, *bufs_and_sems):
        bufs = bufs_and_sems[:nbuf]
        sem_g = bufs_and_sems[nbuf:2 * nbuf]
        sem_o = bufs_and_sems[2 * nbuf:]
        wid = lax.axis_index("s") * nc + lax.axis_index("c")
        brow = wid // w_per_b
        scol = (wid % w_per_b) * rows_per_w
        pltpu.sync_copy(idx_hbm.at[brow, pl.ds(scol, rows_per_w)], idx_v)

        def gather_copy(c, p):
            return pltpu.make_async_copy(
                table_hbm.at[idx_v.at[pl.ds(c * chunk, chunk)]],
                bufs[p], sem_g[p])

        def out_copy(c, p):
            return pltpu.make_async_copy(
                bufs[p], out_hbm.at[brow, pl.ds(scol + c * chunk, chunk)],
                sem_o[p])

        for b in range(nbuf):
            gather_copy(b, b).start()

        def round_body(i, carry):
            g0 = i * nbuf
            for b in range(nbuf):
                c = g0 + b
                gather_copy(c, b).wait()
                out_copy(c, b).start()
                out_copy(c, b).wait()
                gather_copy(c + nbuf, b).start()
            return carry

        lax.fori_loop(0, nround - 1, round_body, 0)

        g0 = (nround - 1) * nbuf
        for b in range(nbuf):
            gather_copy(g0 + b, b).wait()
            out_copy(g0 + b, b).start()
        for b in range(nbuf):
            out_copy(g0 + b, b).wait()

    return gather


def kernel(x, emb_weight):
    return _build_gather()(x.astype(jnp.int32), emb_weight)
